# baseline (speedup 1.0000x reference)
import functools

import jax
import jax.numpy as jnp
from jax import lax
from jax.experimental import pallas as pl
from jax.experimental.pallas import tpu as pltpu
from jax.experimental.pallas import tpu_sc as plsc

BATCH = 4096
HIST = 20
DIM = 64
TOTAL = BATCH * HIST
NUM_CORES = 2
NUM_SUBCORES = 16
NW = NUM_CORES * NUM_SUBCORES
PER_W = TOTAL // NW
CHUNK = 128
N_CHUNK = PER_W // CHUNK

_mesh = plsc.VectorSubcoreMesh(core_axis_name="c", subcore_axis_name="s")


@functools.partial(
    pl.kernel,
    mesh=_mesh,
    out_type=jax.ShapeDtypeStruct((TOTAL, DIM), jnp.float32),
    scratch_types=[
        pltpu.VMEM((N_CHUNK, CHUNK), jnp.int32),
        pltpu.VMEM((CHUNK, DIM), jnp.float32),
        pltpu.SemaphoreType.DMA,
    ],
    compiler_params=pltpu.CompilerParams(use_tc_tiling_on_sc=False),
)
def _gather_kernel(idx_hbm, table_hbm, out_hbm, idx_v, rows_v, sem):
    wid = lax.axis_index("s") * NUM_CORES + lax.axis_index("c")
    base = wid * PER_W
    pltpu.sync_copy(idx_hbm.at[wid], idx_v)
    for j in range(N_CHUNK):
        pltpu.async_copy(table_hbm.at[idx_v.at[j]], rows_v, sem).wait()
        pltpu.sync_copy(rows_v, out_hbm.at[pl.ds(base + j * CHUNK, CHUNK)])


def kernel(indices, table):
    idx = indices.astype(jnp.int32).reshape(NW, N_CHUNK, CHUNK)
    out = _gather_kernel(idx, table)
    return out.reshape(BATCH, HIST, DIM)


# speedup vs baseline: 1.0142x; 1.0142x over previous
import functools

import jax
import jax.numpy as jnp
from jax import lax
from jax.experimental import pallas as pl
from jax.experimental.pallas import tpu as pltpu
from jax.experimental.pallas import tpu_sc as plsc

BATCH = 4096
HIST = 20
DIM = 64
TOTAL = BATCH * HIST
NUM_CORES = 2
NUM_SUBCORES = 16
NW = NUM_CORES * NUM_SUBCORES
PER_W = TOTAL // NW
CHUNK = 128
N_CHUNK = PER_W // CHUNK

GROUP = 5
N_GROUP = N_CHUNK // GROUP
GROUP_ROWS = GROUP * CHUNK

_mesh = plsc.VectorSubcoreMesh(core_axis_name="c", subcore_axis_name="s")


@functools.partial(
    pl.kernel,
    mesh=_mesh,
    out_type=jax.ShapeDtypeStruct((TOTAL, DIM), jnp.float32),
    scratch_types=[
        pltpu.VMEM((N_CHUNK, CHUNK), jnp.int32),
        pltpu.VMEM((2, GROUP_ROWS, DIM), jnp.float32),
        pltpu.SemaphoreType.DMA,
        pltpu.SemaphoreType.DMA,
        pltpu.SemaphoreType.DMA,
        pltpu.SemaphoreType.DMA,
    ],
    compiler_params=pltpu.CompilerParams(use_tc_tiling_on_sc=False),
)
def _gather_kernel(idx_hbm, table_hbm, out_hbm, idx_v, buf, sg0, sg1, so0, so1):
    wid = lax.axis_index("s") * NUM_CORES + lax.axis_index("c")
    base = wid * PER_W
    sem_g = [sg0, sg1]
    sem_o = [so0, so1]
    pltpu.sync_copy(idx_hbm.at[wid], idx_v)

    gathers = [[], []]
    out_copies = [None, None]

    def fire_group(g):
        b = g % 2
        for k in range(GROUP):
            j = g * GROUP + k
            gathers[b].append(pltpu.async_copy(
                table_hbm.at[idx_v.at[j]],
                buf.at[b, pl.ds(k * CHUNK, CHUNK)],
                sem_g[b]))

    fire_group(0)
    for g in range(N_GROUP):
        b = g % 2
        for c in gathers[b]:
            c.wait()
        gathers[b] = []
        if g + 1 < N_GROUP:
            nb = (g + 1) % 2
            if out_copies[nb] is not None:
                out_copies[nb].wait()
                out_copies[nb] = None
            fire_group(g + 1)
        out_copies[b] = pltpu.async_copy(
            buf.at[b],
            out_hbm.at[pl.ds(base + g * GROUP_ROWS, GROUP_ROWS)],
            sem_o[b])
    for b in range(2):
        if out_copies[b] is not None:
            out_copies[b].wait()


def kernel(indices, table):
    idx = indices.astype(jnp.int32).reshape(NW, N_CHUNK, CHUNK)
    out = _gather_kernel(idx, table)
    return out.reshape(BATCH, HIST, DIM)
